# Initial kernel scaffold; baseline (speedup 1.0000x reference)
import jax
import jax.numpy as jnp
from jax.experimental import pallas as pl


def kernel(inp, edge_index, weight1, gcn1_w, gcn2_w, weight2):
    raise NotImplementedError("write your pallas kernel here")


# speedup vs baseline: 4.4013x; 4.4013x over previous
import functools

import jax
import jax.numpy as jnp
from jax import lax
from jax.experimental import pallas as pl
from jax.experimental.pallas import tpu as pltpu
from jax.experimental.pallas import tpu_sc as plsc

_N = 10000
_E = 160000
_H = 256
_HH = 128
_OUT = 128

_NSUB = 16
_PER_SUB_E = _E // _NSUB
_CK = 125
_CHUNKS = _PER_SUB_E // _CK
_NPAD = 10240
_ROWS_PER_SUB = _NPAD // _NSUB

_BM = 256


_sc_mesh = plsc.VectorSubcoreMesh(core_axis_name="c", subcore_axis_name="s")


@functools.partial(
    pl.kernel,
    out_type=jax.ShapeDtypeStruct((2, _NPAD, _HH), jnp.float32),
    mesh=_sc_mesh,
    scratch_types=[
        pltpu.VMEM((_CHUNKS, _CK), jnp.int32),
        pltpu.VMEM((_CHUNKS, _CK), jnp.int32),
        pltpu.VMEM((_CK, _HH), jnp.float32),
        pltpu.VMEM_SHARED((_NPAD, _HH), jnp.float32),
    ],
)
def _adj_spmm_sc(sup0, sup1, src_hbm, dst_hbm, zeros_hbm, out_hbm,
                 src_v, dst_v, rows_v, acc_sh):
    sid = lax.axis_index("s")
    cid = lax.axis_index("c")
    pltpu.sync_copy(src_hbm.at[sid], src_v)
    pltpu.sync_copy(dst_hbm.at[sid], dst_v)
    row0 = sid * _ROWS_PER_SUB
    pltpu.sync_copy(zeros_hbm, acc_sh.at[pl.ds(row0, _ROWS_PER_SUB)])
    plsc.subcore_barrier()

    def run(sup, out_half):
        def body(j, carry):
            pltpu.sync_copy(sup.at[src_v.at[j]], rows_v)
            pltpu.sync_copy(rows_v, acc_sh.at[dst_v.at[j]], add=True)
            return carry

        lax.fori_loop(0, _CHUNKS, body, 0)
        plsc.subcore_barrier()
        pltpu.sync_copy(acc_sh.at[pl.ds(row0, _ROWS_PER_SUB)],
                        out_half.at[pl.ds(row0, _ROWS_PER_SUB)])

    @pl.when(cid == 0)
    def _():
        run(sup0, out_hbm.at[0])

    @pl.when(cid == 1)
    def _():
        run(sup1, out_hbm.at[1])


def _stage1_body(a_ref, w1_ref, g1_ref, x_ref, s1a_ref, s1b_ref):
    x = jnp.dot(a_ref[...], w1_ref[...], preferred_element_type=jnp.float32)
    x_ref[...] = x
    g1 = g1_ref[...]
    s1a_ref[...] = jnp.dot(x, g1[:, :_HH], preferred_element_type=jnp.float32)
    s1b_ref[...] = jnp.dot(x, g1[:, _HH:], preferred_element_type=jnp.float32)


def _stage2_body(ga_ref, gb_ref, g2_ref, g1t_ref, s2a_ref, s2b_ref):
    ga = jnp.tanh(ga_ref[...])
    gb = jnp.tanh(gb_ref[...])
    g2 = g2_ref[...]
    s2a_ref[...] = (jnp.dot(ga, g2[:_HH, :_HH], preferred_element_type=jnp.float32)
                    + jnp.dot(gb, g2[_HH:, :_HH], preferred_element_type=jnp.float32))
    s2b_ref[...] = (jnp.dot(ga, g2[:_HH, _HH:], preferred_element_type=jnp.float32)
                    + jnp.dot(gb, g2[_HH:, _HH:], preferred_element_type=jnp.float32))
    g1t_ref[...] = jnp.concatenate([ga, gb], axis=1)


def _stage3_body(x_ref, g1t_ref, g2a_ref, g2b_ref, w2_ref, o_ref):
    w2 = w2_ref[...]
    acc = jnp.dot(x_ref[...], w2[:_H], preferred_element_type=jnp.float32)
    acc = acc + jnp.dot(g1t_ref[...], w2[_H:2 * _H],
                        preferred_element_type=jnp.float32)
    acc = acc + jnp.dot(g2a_ref[...], w2[2 * _H:2 * _H + _HH],
                        preferred_element_type=jnp.float32)
    acc = acc + jnp.dot(g2b_ref[...], w2[2 * _H + _HH:],
                        preferred_element_type=jnp.float32)
    o_ref[...] = acc


def _mblocks():
    return (_N + _BM - 1) // _BM


def _stage1(inp, weight1, gcn1_w):
    return pl.pallas_call(
        _stage1_body,
        grid=(_mblocks(),),
        in_specs=[
            pl.BlockSpec((_BM, _N), lambda i: (i, 0)),
            pl.BlockSpec((_N, _H), lambda i: (0, 0)),
            pl.BlockSpec((_H, _H), lambda i: (0, 0)),
        ],
        out_specs=[
            pl.BlockSpec((_BM, _H), lambda i: (i, 0)),
            pl.BlockSpec((_BM, _HH), lambda i: (i, 0)),
            pl.BlockSpec((_BM, _HH), lambda i: (i, 0)),
        ],
        out_shape=[
            jax.ShapeDtypeStruct((_N, _H), jnp.float32),
            jax.ShapeDtypeStruct((_N, _HH), jnp.float32),
            jax.ShapeDtypeStruct((_N, _HH), jnp.float32),
        ],
    )(inp, weight1, gcn1_w)


def _stage2(g1a, g1b, gcn2_w):
    return pl.pallas_call(
        _stage2_body,
        grid=(_mblocks(),),
        in_specs=[
            pl.BlockSpec((_BM, _HH), lambda i: (i, 0)),
            pl.BlockSpec((_BM, _HH), lambda i: (i, 0)),
            pl.BlockSpec((_H, _H), lambda i: (0, 0)),
        ],
        out_specs=[
            pl.BlockSpec((_BM, _H), lambda i: (i, 0)),
            pl.BlockSpec((_BM, _HH), lambda i: (i, 0)),
            pl.BlockSpec((_BM, _HH), lambda i: (i, 0)),
        ],
        out_shape=[
            jax.ShapeDtypeStruct((_N, _H), jnp.float32),
            jax.ShapeDtypeStruct((_N, _HH), jnp.float32),
            jax.ShapeDtypeStruct((_N, _HH), jnp.float32),
        ],
    )(g1a, g1b, gcn2_w)


def _stage3(x, g1t, g2a, g2b, weight2):
    return pl.pallas_call(
        _stage3_body,
        grid=(_mblocks(),),
        in_specs=[
            pl.BlockSpec((_BM, _H), lambda i: (i, 0)),
            pl.BlockSpec((_BM, _H), lambda i: (i, 0)),
            pl.BlockSpec((_BM, _HH), lambda i: (i, 0)),
            pl.BlockSpec((_BM, _HH), lambda i: (i, 0)),
            pl.BlockSpec((3 * _H, _OUT), lambda i: (0, 0)),
        ],
        out_specs=pl.BlockSpec((_BM, _OUT), lambda i: (i, 0)),
        out_shape=jax.ShapeDtypeStruct((_N, _OUT), jnp.float32),
    )(x, g1t, g2a, g2b, weight2)


def kernel(inp, edge_index, weight1, gcn1_w, gcn2_w, weight2):
    src = edge_index[0].astype(jnp.int32).reshape(_NSUB, _CHUNKS, _CK)
    dst = edge_index[1].astype(jnp.int32).reshape(_NSUB, _CHUNKS, _CK)
    zeros = jnp.zeros((_ROWS_PER_SUB, _HH), jnp.float32)

    x, s1a, s1b = _stage1(inp, weight1, gcn1_w)
    gnn1 = _adj_spmm_sc(s1a, s1b, src, dst, zeros)
    g1t, s2a, s2b = _stage2(gnn1[0, :_N], gnn1[1, :_N], gcn2_w)
    gnn2 = _adj_spmm_sc(s2a, s2b, src, dst, zeros)
    return _stage3(x, g1t, gnn2[0, :_N], gnn2[1, :_N], weight2)


# speedup vs baseline: 5.1288x; 1.1653x over previous
import functools

import jax
import jax.numpy as jnp
from jax import lax
from jax.experimental import pallas as pl
from jax.experimental.pallas import tpu as pltpu
from jax.experimental.pallas import tpu_sc as plsc

_N = 10000
_E = 160000
_H = 256
_HH = 128
_OUT = 128

_NSUB = 16
_PER_SUB_E = _E // _NSUB
_CK = 125
_CHUNKS = _PER_SUB_E // _CK
_G = 16
_NG = _CHUNKS // _G
_NPAD = 10112
_ROWS_PER_SUB = _NPAD // _NSUB

_BM = 256


_sc_mesh = plsc.VectorSubcoreMesh(core_axis_name="c", subcore_axis_name="s")


@functools.partial(
    pl.kernel,
    out_type=jax.ShapeDtypeStruct((2, _NPAD, _HH), jnp.float32),
    mesh=_sc_mesh,
    scratch_types=[
        pltpu.VMEM((2, _G, _CK), jnp.int32),
        pltpu.VMEM((2, _G, _CK), jnp.int32),
        pltpu.VMEM((_CK, _HH), jnp.float32),
        pltpu.VMEM((_CK, _HH), jnp.float32),
        pltpu.VMEM_SHARED((_NPAD, _HH), jnp.float32),
        pltpu.SemaphoreType.DMA,
        pltpu.SemaphoreType.DMA,
        pltpu.SemaphoreType.DMA,
    ],
)
def _adj_spmm_sc(sup0, sup1, src_hbm, dst_hbm, zeros_hbm, out_hbm,
                 src_g, dst_g, rows0_v, rows1_v, acc_sh, sem0, sem1, semi):
    sid = lax.axis_index("s")
    cid = lax.axis_index("c")
    src_h = src_hbm.at[sid]
    dst_h = dst_hbm.at[sid]
    row0 = sid * _ROWS_PER_SUB
    pltpu.sync_copy(zeros_hbm, acc_sh.at[pl.ds(row0, _ROWS_PER_SUB)])
    plsc.subcore_barrier()

    def run(sup, out_half):
        pltpu.sync_copy(src_h.at[pl.ds(0, _G)], src_g.at[0])
        pltpu.sync_copy(dst_h.at[pl.ds(0, _G)], dst_g.at[0])
        pltpu.async_copy(sup.at[src_g.at[0].at[0]], rows0_v, sem0)

        for g in range(_NG):
            b = g % 2
            nb = (g + 1) % 2
            if g + 1 < _NG:
                cpi_s = pltpu.async_copy(
                    src_h.at[pl.ds((g + 1) * _G, _G)], src_g.at[nb], semi)
                cpi_d = pltpu.async_copy(
                    dst_h.at[pl.ds((g + 1) * _G, _G)], dst_g.at[nb], semi)

            def body(i, carry, b=b):
                k0 = 2 * i
                pltpu.make_async_copy(
                    sup.at[src_g.at[b].at[k0]], rows0_v, sem0).wait()
                pltpu.async_copy(
                    sup.at[src_g.at[b].at[k0 + 1]], rows1_v, sem1)
                pltpu.sync_copy(rows0_v, acc_sh.at[dst_g.at[b].at[k0]],
                                add=True)
                pltpu.make_async_copy(
                    sup.at[src_g.at[b].at[k0 + 1]], rows1_v, sem1).wait()

                @pl.when(k0 + 2 < _G)
                def _():
                    pltpu.async_copy(
                        sup.at[src_g.at[b].at[k0 + 2]], rows0_v, sem0)

                pltpu.sync_copy(rows1_v, acc_sh.at[dst_g.at[b].at[k0 + 1]],
                                add=True)
                return carry

            lax.fori_loop(0, _G // 2, body, 0)
            if g + 1 < _NG:
                cpi_s.wait()
                cpi_d.wait()
                pltpu.async_copy(sup.at[src_g.at[nb].at[0]], rows0_v, sem0)

        plsc.subcore_barrier()
        pltpu.sync_copy(acc_sh.at[pl.ds(row0, _ROWS_PER_SUB)],
                        out_half.at[pl.ds(row0, _ROWS_PER_SUB)])

    @pl.when(cid == 0)
    def _():
        run(sup0, out_hbm.at[0])

    @pl.when(cid == 1)
    def _():
        run(sup1, out_hbm.at[1])


def _stage1_body(a_ref, w1_ref, g1_ref, x_ref, s1a_ref, s1b_ref):
    x = jnp.dot(a_ref[...], w1_ref[...], preferred_element_type=jnp.float32)
    x_ref[...] = x
    g1 = g1_ref[...]
    s1a_ref[...] = jnp.dot(x, g1[:, :_HH], preferred_element_type=jnp.float32)
    s1b_ref[...] = jnp.dot(x, g1[:, _HH:], preferred_element_type=jnp.float32)


def _stage2_body(ga_ref, gb_ref, g2_ref, g1t_ref, s2a_ref, s2b_ref):
    ga = jnp.tanh(ga_ref[...])
    gb = jnp.tanh(gb_ref[...])
    g2 = g2_ref[...]
    s2a_ref[...] = (jnp.dot(ga, g2[:_HH, :_HH], preferred_element_type=jnp.float32)
                    + jnp.dot(gb, g2[_HH:, :_HH], preferred_element_type=jnp.float32))
    s2b_ref[...] = (jnp.dot(ga, g2[:_HH, _HH:], preferred_element_type=jnp.float32)
                    + jnp.dot(gb, g2[_HH:, _HH:], preferred_element_type=jnp.float32))
    g1t_ref[...] = jnp.concatenate([ga, gb], axis=1)


def _stage3_body(x_ref, g1t_ref, g2a_ref, g2b_ref, w2_ref, o_ref):
    w2 = w2_ref[...]
    acc = jnp.dot(x_ref[...], w2[:_H], preferred_element_type=jnp.float32)
    acc = acc + jnp.dot(g1t_ref[...], w2[_H:2 * _H],
                        preferred_element_type=jnp.float32)
    acc = acc + jnp.dot(g2a_ref[...], w2[2 * _H:2 * _H + _HH],
                        preferred_element_type=jnp.float32)
    acc = acc + jnp.dot(g2b_ref[...], w2[2 * _H + _HH:],
                        preferred_element_type=jnp.float32)
    o_ref[...] = acc


def _mblocks():
    return (_N + _BM - 1) // _BM


def _stage1(inp, weight1, gcn1_w):
    return pl.pallas_call(
        _stage1_body,
        grid=(_mblocks(),),
        in_specs=[
            pl.BlockSpec((_BM, _N), lambda i: (i, 0)),
            pl.BlockSpec((_N, _H), lambda i: (0, 0)),
            pl.BlockSpec((_H, _H), lambda i: (0, 0)),
        ],
        out_specs=[
            pl.BlockSpec((_BM, _H), lambda i: (i, 0)),
            pl.BlockSpec((_BM, _HH), lambda i: (i, 0)),
            pl.BlockSpec((_BM, _HH), lambda i: (i, 0)),
        ],
        out_shape=[
            jax.ShapeDtypeStruct((_N, _H), jnp.float32),
            jax.ShapeDtypeStruct((_N, _HH), jnp.float32),
            jax.ShapeDtypeStruct((_N, _HH), jnp.float32),
        ],
    )(inp, weight1, gcn1_w)


def _stage2(g1a, g1b, gcn2_w):
    return pl.pallas_call(
        _stage2_body,
        grid=(_mblocks(),),
        in_specs=[
            pl.BlockSpec((_BM, _HH), lambda i: (i, 0)),
            pl.BlockSpec((_BM, _HH), lambda i: (i, 0)),
            pl.BlockSpec((_H, _H), lambda i: (0, 0)),
        ],
        out_specs=[
            pl.BlockSpec((_BM, _H), lambda i: (i, 0)),
            pl.BlockSpec((_BM, _HH), lambda i: (i, 0)),
            pl.BlockSpec((_BM, _HH), lambda i: (i, 0)),
        ],
        out_shape=[
            jax.ShapeDtypeStruct((_N, _H), jnp.float32),
            jax.ShapeDtypeStruct((_N, _HH), jnp.float32),
            jax.ShapeDtypeStruct((_N, _HH), jnp.float32),
        ],
    )(g1a, g1b, gcn2_w)


def _stage3(x, g1t, g2a, g2b, weight2):
    return pl.pallas_call(
        _stage3_body,
        grid=(_mblocks(),),
        in_specs=[
            pl.BlockSpec((_BM, _H), lambda i: (i, 0)),
            pl.BlockSpec((_BM, _H), lambda i: (i, 0)),
            pl.BlockSpec((_BM, _HH), lambda i: (i, 0)),
            pl.BlockSpec((_BM, _HH), lambda i: (i, 0)),
            pl.BlockSpec((3 * _H, _OUT), lambda i: (0, 0)),
        ],
        out_specs=pl.BlockSpec((_BM, _OUT), lambda i: (i, 0)),
        out_shape=jax.ShapeDtypeStruct((_N, _OUT), jnp.float32),
    )(x, g1t, g2a, g2b, weight2)


def kernel(inp, edge_index, weight1, gcn1_w, gcn2_w, weight2):
    src = edge_index[0].astype(jnp.int32).reshape(_NSUB, _CHUNKS, _CK)
    dst = edge_index[1].astype(jnp.int32).reshape(_NSUB, _CHUNKS, _CK)
    zeros = jnp.zeros((_ROWS_PER_SUB, _HH), jnp.float32)

    x, s1a, s1b = _stage1(inp, weight1, gcn1_w)
    gnn1 = _adj_spmm_sc(s1a, s1b, src, dst, zeros)
    g1t, s2a, s2b = _stage2(gnn1[0, :_N], gnn1[1, :_N], gcn2_w)
    gnn2 = _adj_spmm_sc(s2a, s2b, src, dst, zeros)
    return _stage3(x, g1t, gnn2[0, :_N], gnn2[1, :_N], weight2)


# speedup vs baseline: 5.1412x; 1.0024x over previous
import functools

import jax
import jax.numpy as jnp
from jax import lax
from jax.experimental import pallas as pl
from jax.experimental.pallas import tpu as pltpu
from jax.experimental.pallas import tpu_sc as plsc

_N = 10000
_E = 160000
_H = 256
_HH = 128
_OUT = 128

_NSUB = 16
_PER_SUB_E = _E // _NSUB
_CK = 125
_CHUNKS = _PER_SUB_E // _CK
_G = 16
_NG = _CHUNKS // _G
_NPAD = 10112
_ROWS_PER_SUB = _NPAD // _NSUB

_BM = 256
_BM1 = 512


_sc_mesh = plsc.VectorSubcoreMesh(core_axis_name="c", subcore_axis_name="s")


@functools.partial(
    pl.kernel,
    out_type=jax.ShapeDtypeStruct((2, _NPAD, _HH), jnp.float32),
    mesh=_sc_mesh,
    scratch_types=[
        pltpu.VMEM((2, _G, _CK), jnp.int32),
        pltpu.VMEM((2, _G, _CK), jnp.int32),
        pltpu.VMEM((_CK, _HH), jnp.float32),
        pltpu.VMEM((_CK, _HH), jnp.float32),
        pltpu.VMEM_SHARED((_NPAD, _HH), jnp.float32),
        pltpu.SemaphoreType.DMA,
        pltpu.SemaphoreType.DMA,
        pltpu.SemaphoreType.DMA,
    ],
)
def _adj_spmm_sc(sup0, sup1, src_hbm, dst_hbm, zeros_hbm, out_hbm,
                 src_g, dst_g, rows0_v, rows1_v, acc_sh, sem0, sem1, semi):
    sid = lax.axis_index("s")
    cid = lax.axis_index("c")
    src_h = src_hbm.at[sid]
    dst_h = dst_hbm.at[sid]
    row0 = sid * _ROWS_PER_SUB
    pltpu.sync_copy(zeros_hbm, acc_sh.at[pl.ds(row0, _ROWS_PER_SUB)])
    plsc.subcore_barrier()

    def run(sup, out_half):
        pltpu.sync_copy(src_h.at[pl.ds(0, _G)], src_g.at[0])
        pltpu.sync_copy(dst_h.at[pl.ds(0, _G)], dst_g.at[0])
        pltpu.async_copy(sup.at[src_g.at[0].at[0]], rows0_v, sem0)

        for g in range(_NG):
            b = g % 2
            nb = (g + 1) % 2
            if g + 1 < _NG:
                cpi_s = pltpu.async_copy(
                    src_h.at[pl.ds((g + 1) * _G, _G)], src_g.at[nb], semi)
                cpi_d = pltpu.async_copy(
                    dst_h.at[pl.ds((g + 1) * _G, _G)], dst_g.at[nb], semi)

            def body(i, carry, b=b):
                k0 = 2 * i
                pltpu.make_async_copy(
                    sup.at[src_g.at[b].at[k0]], rows0_v, sem0).wait()
                pltpu.async_copy(
                    sup.at[src_g.at[b].at[k0 + 1]], rows1_v, sem1)
                pltpu.sync_copy(rows0_v, acc_sh.at[dst_g.at[b].at[k0]],
                                add=True)
                pltpu.make_async_copy(
                    sup.at[src_g.at[b].at[k0 + 1]], rows1_v, sem1).wait()

                @pl.when(k0 + 2 < _G)
                def _():
                    pltpu.async_copy(
                        sup.at[src_g.at[b].at[k0 + 2]], rows0_v, sem0)

                pltpu.sync_copy(rows1_v, acc_sh.at[dst_g.at[b].at[k0 + 1]],
                                add=True)
                return carry

            lax.fori_loop(0, _G // 2, body, 0)
            if g + 1 < _NG:
                cpi_s.wait()
                cpi_d.wait()
                pltpu.async_copy(sup.at[src_g.at[nb].at[0]], rows0_v, sem0)

        plsc.subcore_barrier()
        pltpu.sync_copy(acc_sh.at[pl.ds(row0, _ROWS_PER_SUB)],
                        out_half.at[pl.ds(row0, _ROWS_PER_SUB)])

    @pl.when(cid == 0)
    def _():
        run(sup0, out_hbm.at[0])

    @pl.when(cid == 1)
    def _():
        run(sup1, out_hbm.at[1])


def _stage1_body(a_ref, w1_ref, g1_ref, x_ref, s1a_ref, s1b_ref):
    x = jnp.dot(a_ref[...], w1_ref[...], preferred_element_type=jnp.float32)
    x_ref[...] = x
    g1 = g1_ref[...]
    s1a_ref[...] = jnp.dot(x, g1[:, :_HH], preferred_element_type=jnp.float32)
    s1b_ref[...] = jnp.dot(x, g1[:, _HH:], preferred_element_type=jnp.float32)


def _stage2_body(ga_ref, gb_ref, g2_ref, g1t_ref, s2a_ref, s2b_ref):
    ga = jnp.tanh(ga_ref[...])
    gb = jnp.tanh(gb_ref[...])
    g2 = g2_ref[...]
    s2a_ref[...] = (jnp.dot(ga, g2[:_HH, :_HH], preferred_element_type=jnp.float32)
                    + jnp.dot(gb, g2[_HH:, :_HH], preferred_element_type=jnp.float32))
    s2b_ref[...] = (jnp.dot(ga, g2[:_HH, _HH:], preferred_element_type=jnp.float32)
                    + jnp.dot(gb, g2[_HH:, _HH:], preferred_element_type=jnp.float32))
    g1t_ref[...] = jnp.concatenate([ga, gb], axis=1)


def _stage3_body(x_ref, g1t_ref, g2a_ref, g2b_ref, w2_ref, o_ref):
    w2 = w2_ref[...]
    acc = jnp.dot(x_ref[...], w2[:_H], preferred_element_type=jnp.float32)
    acc = acc + jnp.dot(g1t_ref[...], w2[_H:2 * _H],
                        preferred_element_type=jnp.float32)
    acc = acc + jnp.dot(g2a_ref[...], w2[2 * _H:2 * _H + _HH],
                        preferred_element_type=jnp.float32)
    acc = acc + jnp.dot(g2b_ref[...], w2[2 * _H + _HH:],
                        preferred_element_type=jnp.float32)
    o_ref[...] = acc


def _mblocks():
    return (_N + _BM - 1) // _BM


def _stage1(inp, weight1, gcn1_w):
    return pl.pallas_call(
        _stage1_body,
        grid=((_N + _BM1 - 1) // _BM1,),
        in_specs=[
            pl.BlockSpec((_BM1, _N), lambda i: (i, 0)),
            pl.BlockSpec((_N, _H), lambda i: (0, 0)),
            pl.BlockSpec((_H, _H), lambda i: (0, 0)),
        ],
        out_specs=[
            pl.BlockSpec((_BM1, _H), lambda i: (i, 0)),
            pl.BlockSpec((_BM1, _HH), lambda i: (i, 0)),
            pl.BlockSpec((_BM1, _HH), lambda i: (i, 0)),
        ],
        out_shape=[
            jax.ShapeDtypeStruct((_N, _H), jnp.float32),
            jax.ShapeDtypeStruct((_N, _HH), jnp.float32),
            jax.ShapeDtypeStruct((_N, _HH), jnp.float32),
        ],
    )(inp, weight1, gcn1_w)


def _stage2(g1a, g1b, gcn2_w):
    return pl.pallas_call(
        _stage2_body,
        grid=(_mblocks(),),
        in_specs=[
            pl.BlockSpec((_BM, _HH), lambda i: (i, 0)),
            pl.BlockSpec((_BM, _HH), lambda i: (i, 0)),
            pl.BlockSpec((_H, _H), lambda i: (0, 0)),
        ],
        out_specs=[
            pl.BlockSpec((_BM, _H), lambda i: (i, 0)),
            pl.BlockSpec((_BM, _HH), lambda i: (i, 0)),
            pl.BlockSpec((_BM, _HH), lambda i: (i, 0)),
        ],
        out_shape=[
            jax.ShapeDtypeStruct((_N, _H), jnp.float32),
            jax.ShapeDtypeStruct((_N, _HH), jnp.float32),
            jax.ShapeDtypeStruct((_N, _HH), jnp.float32),
        ],
    )(g1a, g1b, gcn2_w)


def _stage3(x, g1t, g2a, g2b, weight2):
    return pl.pallas_call(
        _stage3_body,
        grid=(_mblocks(),),
        in_specs=[
            pl.BlockSpec((_BM, _H), lambda i: (i, 0)),
            pl.BlockSpec((_BM, _H), lambda i: (i, 0)),
            pl.BlockSpec((_BM, _HH), lambda i: (i, 0)),
            pl.BlockSpec((_BM, _HH), lambda i: (i, 0)),
            pl.BlockSpec((3 * _H, _OUT), lambda i: (0, 0)),
        ],
        out_specs=pl.BlockSpec((_BM, _OUT), lambda i: (i, 0)),
        out_shape=jax.ShapeDtypeStruct((_N, _OUT), jnp.float32),
    )(x, g1t, g2a, g2b, weight2)


def kernel(inp, edge_index, weight1, gcn1_w, gcn2_w, weight2):
    src = edge_index[0].astype(jnp.int32).reshape(_NSUB, _CHUNKS, _CK)
    dst = edge_index[1].astype(jnp.int32).reshape(_NSUB, _CHUNKS, _CK)
    zeros = jnp.zeros((_ROWS_PER_SUB, _HH), jnp.float32)

    x, s1a, s1b = _stage1(inp, weight1, gcn1_w)
    gnn1 = _adj_spmm_sc(s1a, s1b, src, dst, zeros)
    g1t, s2a, s2b = _stage2(gnn1[0, :_N], gnn1[1, :_N], gcn2_w)
    gnn2 = _adj_spmm_sc(s2a, s2b, src, dst, zeros)
    return _stage3(x, g1t, gnn2[0, :_N], gnn2[1, :_N], weight2)


# speedup vs baseline: 5.1729x; 1.0062x over previous
import functools

import jax
import jax.numpy as jnp
from jax import lax
from jax.experimental import pallas as pl
from jax.experimental.pallas import tpu as pltpu
from jax.experimental.pallas import tpu_sc as plsc

_N = 10000
_E = 160000
_H = 256
_HH = 128
_OUT = 128

_NSUB = 16
_PER_SUB_E = _E // _NSUB
_CK = 125
_CHUNKS = _PER_SUB_E // _CK
_G = 16
_NG = _CHUNKS // _G
_NPAD = 10112
_ROWS_PER_SUB = _NPAD // _NSUB

_BM = 256
_BM1 = 512


_sc_mesh = plsc.VectorSubcoreMesh(core_axis_name="c", subcore_axis_name="s")


@functools.partial(
    pl.kernel,
    out_type=jax.ShapeDtypeStruct((2, _NPAD, _HH), jnp.float32),
    mesh=_sc_mesh,
    scratch_types=[
        pltpu.VMEM((2, _G, _CK), jnp.int32),
        pltpu.VMEM((2, _G, _CK), jnp.int32),
        pltpu.VMEM((_CK, _HH), jnp.float32),
        pltpu.VMEM((_CK, _HH), jnp.float32),
        pltpu.VMEM_SHARED((_NPAD, _HH), jnp.float32),
        pltpu.SemaphoreType.DMA,
        pltpu.SemaphoreType.DMA,
        pltpu.SemaphoreType.DMA,
    ],
)
def _adj_spmm_sc(sup0, sup1, src_hbm, dst_hbm, zeros_hbm, out_hbm,
                 src_g, dst_g, rows0_v, rows1_v, acc_sh, sem0, sem1, semi):
    sid = lax.axis_index("s")
    cid = lax.axis_index("c")
    src_h = src_hbm.at[sid]
    dst_h = dst_hbm.at[sid]
    row0 = sid * _ROWS_PER_SUB
    pltpu.sync_copy(zeros_hbm, acc_sh.at[pl.ds(row0, _ROWS_PER_SUB)])
    plsc.subcore_barrier()

    def run(sup, out_half):
        pltpu.sync_copy(src_h.at[pl.ds(0, _G)], src_g.at[0])
        pltpu.sync_copy(dst_h.at[pl.ds(0, _G)], dst_g.at[0])
        pltpu.async_copy(sup.at[src_g.at[0].at[0]], rows0_v, sem0)

        for g in range(_NG):
            b = g % 2
            nb = (g + 1) % 2
            if g + 1 < _NG:
                cpi_s = pltpu.async_copy(
                    src_h.at[pl.ds((g + 1) * _G, _G)], src_g.at[nb], semi)
                cpi_d = pltpu.async_copy(
                    dst_h.at[pl.ds((g + 1) * _G, _G)], dst_g.at[nb], semi)

            def body(i, carry, b=b):
                k0 = 2 * i
                pltpu.make_async_copy(
                    sup.at[src_g.at[b].at[k0]], rows0_v, sem0).wait()
                pltpu.async_copy(
                    sup.at[src_g.at[b].at[k0 + 1]], rows1_v, sem1)
                pltpu.sync_copy(rows0_v, acc_sh.at[dst_g.at[b].at[k0]],
                                add=True)
                pltpu.make_async_copy(
                    sup.at[src_g.at[b].at[k0 + 1]], rows1_v, sem1).wait()

                @pl.when(k0 + 2 < _G)
                def _():
                    pltpu.async_copy(
                        sup.at[src_g.at[b].at[k0 + 2]], rows0_v, sem0)

                pltpu.sync_copy(rows1_v, acc_sh.at[dst_g.at[b].at[k0 + 1]],
                                add=True)
                return carry

            lax.fori_loop(0, _G // 2, body, 0)
            if g + 1 < _NG:
                cpi_s.wait()
                cpi_d.wait()
                pltpu.async_copy(sup.at[src_g.at[nb].at[0]], rows0_v, sem0)

        plsc.subcore_barrier()
        pltpu.sync_copy(acc_sh.at[pl.ds(row0, _ROWS_PER_SUB)],
                        out_half.at[pl.ds(row0, _ROWS_PER_SUB)])

    @pl.when(cid == 0)
    def _():
        run(sup0, out_hbm.at[0])

    @pl.when(cid == 1)
    def _():
        run(sup1, out_hbm.at[1])


def _stage1_body(a_ref, w1_ref, g1_ref, x_ref, s1a_ref, s1b_ref):
    x = jnp.dot(a_ref[...], w1_ref[...], preferred_element_type=jnp.float32)
    x_ref[...] = x
    g1 = g1_ref[...]
    s1a_ref[...] = jnp.dot(x, g1[:, :_HH], preferred_element_type=jnp.float32)
    s1b_ref[...] = jnp.dot(x, g1[:, _HH:], preferred_element_type=jnp.float32)


def _stage2_body(ga_ref, gb_ref, g2_ref, g1t_ref, s2a_ref, s2b_ref):
    ga = jnp.tanh(ga_ref[...])
    gb = jnp.tanh(gb_ref[...])
    g2 = g2_ref[...]
    s2a_ref[...] = (jnp.dot(ga, g2[:_HH, :_HH], preferred_element_type=jnp.float32)
                    + jnp.dot(gb, g2[_HH:, :_HH], preferred_element_type=jnp.float32))
    s2b_ref[...] = (jnp.dot(ga, g2[:_HH, _HH:], preferred_element_type=jnp.float32)
                    + jnp.dot(gb, g2[_HH:, _HH:], preferred_element_type=jnp.float32))
    g1t_ref[...] = jnp.concatenate([ga, gb], axis=1)


def _stage3a_body(x_ref, g1t_ref, w2_ref, p_ref):
    w2 = w2_ref[...]
    p_ref[...] = (jnp.dot(x_ref[...], w2[:_H],
                          preferred_element_type=jnp.float32)
                  + jnp.dot(g1t_ref[...], w2[_H:2 * _H],
                            preferred_element_type=jnp.float32))


def _stage3b_body(p_ref, g2a_ref, g2b_ref, w2_ref, o_ref):
    w2 = w2_ref[...]
    acc = p_ref[...]
    acc = acc + jnp.dot(g2a_ref[...], w2[2 * _H:2 * _H + _HH],
                        preferred_element_type=jnp.float32)
    acc = acc + jnp.dot(g2b_ref[...], w2[2 * _H + _HH:],
                        preferred_element_type=jnp.float32)
    o_ref[...] = acc


def _mblocks():
    return (_N + _BM - 1) // _BM


def _stage1(inp, weight1, gcn1_w):
    return pl.pallas_call(
        _stage1_body,
        grid=((_N + _BM1 - 1) // _BM1,),
        in_specs=[
            pl.BlockSpec((_BM1, _N), lambda i: (i, 0)),
            pl.BlockSpec((_N, _H), lambda i: (0, 0)),
            pl.BlockSpec((_H, _H), lambda i: (0, 0)),
        ],
        out_specs=[
            pl.BlockSpec((_BM1, _H), lambda i: (i, 0)),
            pl.BlockSpec((_BM1, _HH), lambda i: (i, 0)),
            pl.BlockSpec((_BM1, _HH), lambda i: (i, 0)),
        ],
        out_shape=[
            jax.ShapeDtypeStruct((_N, _H), jnp.float32),
            jax.ShapeDtypeStruct((_N, _HH), jnp.float32),
            jax.ShapeDtypeStruct((_N, _HH), jnp.float32),
        ],
    )(inp, weight1, gcn1_w)


def _stage2(g1a, g1b, gcn2_w):
    return pl.pallas_call(
        _stage2_body,
        grid=(_mblocks(),),
        in_specs=[
            pl.BlockSpec((_BM, _HH), lambda i: (i, 0)),
            pl.BlockSpec((_BM, _HH), lambda i: (i, 0)),
            pl.BlockSpec((_H, _H), lambda i: (0, 0)),
        ],
        out_specs=[
            pl.BlockSpec((_BM, _H), lambda i: (i, 0)),
            pl.BlockSpec((_BM, _HH), lambda i: (i, 0)),
            pl.BlockSpec((_BM, _HH), lambda i: (i, 0)),
        ],
        out_shape=[
            jax.ShapeDtypeStruct((_N, _H), jnp.float32),
            jax.ShapeDtypeStruct((_N, _HH), jnp.float32),
            jax.ShapeDtypeStruct((_N, _HH), jnp.float32),
        ],
    )(g1a, g1b, gcn2_w)


def _stage3a(x, g1t, weight2):
    return pl.pallas_call(
        _stage3a_body,
        grid=(_mblocks(),),
        in_specs=[
            pl.BlockSpec((_BM, _H), lambda i: (i, 0)),
            pl.BlockSpec((_BM, _H), lambda i: (i, 0)),
            pl.BlockSpec((3 * _H, _OUT), lambda i: (0, 0)),
        ],
        out_specs=pl.BlockSpec((_BM, _OUT), lambda i: (i, 0)),
        out_shape=jax.ShapeDtypeStruct((_N, _OUT), jnp.float32),
    )(x, g1t, weight2)


def _stage3b(p, g2a, g2b, weight2):
    return pl.pallas_call(
        _stage3b_body,
        grid=(_mblocks(),),
        in_specs=[
            pl.BlockSpec((_BM, _OUT), lambda i: (i, 0)),
            pl.BlockSpec((_BM, _HH), lambda i: (i, 0)),
            pl.BlockSpec((_BM, _HH), lambda i: (i, 0)),
            pl.BlockSpec((3 * _H, _OUT), lambda i: (0, 0)),
        ],
        out_specs=pl.BlockSpec((_BM, _OUT), lambda i: (i, 0)),
        out_shape=jax.ShapeDtypeStruct((_N, _OUT), jnp.float32),
    )(p, g2a, g2b, weight2)


def kernel(inp, edge_index, weight1, gcn1_w, gcn2_w, weight2):
    src = edge_index[0].astype(jnp.int32).reshape(_NSUB, _CHUNKS, _CK)
    dst = edge_index[1].astype(jnp.int32).reshape(_NSUB, _CHUNKS, _CK)
    zeros = jnp.zeros((_ROWS_PER_SUB, _HH), jnp.float32)

    x, s1a, s1b = _stage1(inp, weight1, gcn1_w)
    gnn1 = _adj_spmm_sc(s1a, s1b, src, dst, zeros)
    g1t, s2a, s2b = _stage2(gnn1[0, :_N], gnn1[1, :_N], gcn2_w)
    gnn2 = _adj_spmm_sc(s2a, s2b, src, dst, zeros)
    p = _stage3a(x, g1t, weight2)
    return _stage3b(p, gnn2[0, :_N], gnn2[1, :_N], weight2)


# speedup vs baseline: 5.2083x; 1.0068x over previous
import functools

import jax
import jax.numpy as jnp
from jax import lax
from jax.experimental import pallas as pl
from jax.experimental.pallas import tpu as pltpu
from jax.experimental.pallas import tpu_sc as plsc

_N = 10000
_E = 160000
_H = 256
_HH = 128
_OUT = 128

_NSUB = 16
_PER_SUB_E = _E // _NSUB
_CK = 125
_CHUNKS = _PER_SUB_E // _CK
_G = 16
_NG = _CHUNKS // _G
_NPAD = 10112
_ROWS_PER_SUB = _NPAD // _NSUB

_BM = 256
_BM1 = 512


_sc_mesh = plsc.VectorSubcoreMesh(core_axis_name="c", subcore_axis_name="s")


@functools.partial(
    pl.kernel,
    out_type=jax.ShapeDtypeStruct((2, _NPAD, _HH), jnp.float32),
    mesh=_sc_mesh,
    scratch_types=[
        pltpu.VMEM((2, _G, _CK), jnp.int32),
        pltpu.VMEM((2, _G, _CK), jnp.int32),
        pltpu.VMEM((_CK, _HH), jnp.float32),
        pltpu.VMEM((_CK, _HH), jnp.float32),
        pltpu.VMEM_SHARED((_NPAD, _HH), jnp.float32),
        pltpu.SemaphoreType.DMA,
        pltpu.SemaphoreType.DMA,
        pltpu.SemaphoreType.DMA,
    ],
)
def _adj_spmm_sc(sup0, sup1, src_hbm, dst_hbm, zeros_hbm, out_hbm,
                 src_g, dst_g, rows0_v, rows1_v, acc_sh, sem0, sem1, semi):
    sid = lax.axis_index("s")
    cid = lax.axis_index("c")
    src_h = src_hbm.at[sid]
    dst_h = dst_hbm.at[sid]
    row0 = sid * _ROWS_PER_SUB
    pltpu.sync_copy(zeros_hbm, acc_sh.at[pl.ds(row0, _ROWS_PER_SUB)])
    plsc.subcore_barrier()

    def run(sup, out_half):
        pltpu.sync_copy(src_h.at[pl.ds(0, _G)], src_g.at[0])
        pltpu.sync_copy(dst_h.at[pl.ds(0, _G)], dst_g.at[0])
        pltpu.async_copy(sup.at[src_g.at[0].at[0]], rows0_v, sem0)

        for g in range(_NG):
            b = g % 2
            nb = (g + 1) % 2
            if g + 1 < _NG:
                cpi_s = pltpu.async_copy(
                    src_h.at[pl.ds((g + 1) * _G, _G)], src_g.at[nb], semi)
                cpi_d = pltpu.async_copy(
                    dst_h.at[pl.ds((g + 1) * _G, _G)], dst_g.at[nb], semi)

            def body(i, carry, b=b):
                k0 = 2 * i
                pltpu.make_async_copy(
                    sup.at[src_g.at[b].at[k0]], rows0_v, sem0).wait()
                pltpu.async_copy(
                    sup.at[src_g.at[b].at[k0 + 1]], rows1_v, sem1)
                pltpu.sync_copy(rows0_v, acc_sh.at[dst_g.at[b].at[k0]],
                                add=True)
                pltpu.make_async_copy(
                    sup.at[src_g.at[b].at[k0 + 1]], rows1_v, sem1).wait()

                @pl.when(k0 + 2 < _G)
                def _():
                    pltpu.async_copy(
                        sup.at[src_g.at[b].at[k0 + 2]], rows0_v, sem0)

                pltpu.sync_copy(rows1_v, acc_sh.at[dst_g.at[b].at[k0 + 1]],
                                add=True)
                return carry

            lax.fori_loop(0, _G // 2, body, 0)
            if g + 1 < _NG:
                cpi_s.wait()
                cpi_d.wait()
                pltpu.async_copy(sup.at[src_g.at[nb].at[0]], rows0_v, sem0)

        plsc.subcore_barrier()
        pltpu.sync_copy(acc_sh.at[pl.ds(row0, _ROWS_PER_SUB)],
                        out_half.at[pl.ds(row0, _ROWS_PER_SUB)])

    @pl.when(cid == 0)
    def _():
        run(sup0, out_hbm.at[0])

    @pl.when(cid == 1)
    def _():
        run(sup1, out_hbm.at[1])


def _stage1_body(a_ref, w1_ref, g1_ref, w2_ref, s1a_ref, s1b_ref, p0_ref):
    x = jnp.dot(a_ref[...], w1_ref[...], preferred_element_type=jnp.float32)
    g1 = g1_ref[...]
    s1a_ref[...] = jnp.dot(x, g1[:, :_HH], preferred_element_type=jnp.float32)
    s1b_ref[...] = jnp.dot(x, g1[:, _HH:], preferred_element_type=jnp.float32)
    p0_ref[...] = jnp.dot(x, w2_ref[...][:_H],
                          preferred_element_type=jnp.float32)


def _stage2_body(ga_ref, gb_ref, g2_ref, w2_ref, p0_ref, s2a_ref, s2b_ref,
                 p_ref):
    ga = jnp.tanh(ga_ref[...])
    gb = jnp.tanh(gb_ref[...])
    g2 = g2_ref[...]
    w2 = w2_ref[...]
    s2a_ref[...] = (jnp.dot(ga, g2[:_HH, :_HH], preferred_element_type=jnp.float32)
                    + jnp.dot(gb, g2[_HH:, :_HH], preferred_element_type=jnp.float32))
    s2b_ref[...] = (jnp.dot(ga, g2[:_HH, _HH:], preferred_element_type=jnp.float32)
                    + jnp.dot(gb, g2[_HH:, _HH:], preferred_element_type=jnp.float32))
    p_ref[...] = (p0_ref[...]
                  + jnp.dot(ga, w2[_H:_H + _HH],
                            preferred_element_type=jnp.float32)
                  + jnp.dot(gb, w2[_H + _HH:2 * _H],
                            preferred_element_type=jnp.float32))


def _stage3b_body(p_ref, g2a_ref, g2b_ref, w2_ref, o_ref):
    w2 = w2_ref[...]
    acc = p_ref[...]
    acc = acc + jnp.dot(g2a_ref[...], w2[2 * _H:2 * _H + _HH],
                        preferred_element_type=jnp.float32)
    acc = acc + jnp.dot(g2b_ref[...], w2[2 * _H + _HH:],
                        preferred_element_type=jnp.float32)
    o_ref[...] = acc


def _mblocks():
    return (_N + _BM - 1) // _BM


def _stage1(inp, weight1, gcn1_w, weight2):
    return pl.pallas_call(
        _stage1_body,
        grid=((_N + _BM1 - 1) // _BM1,),
        in_specs=[
            pl.BlockSpec((_BM1, _N), lambda i: (i, 0)),
            pl.BlockSpec((_N, _H), lambda i: (0, 0)),
            pl.BlockSpec((_H, _H), lambda i: (0, 0)),
            pl.BlockSpec((3 * _H, _OUT), lambda i: (0, 0)),
        ],
        out_specs=[
            pl.BlockSpec((_BM1, _HH), lambda i: (i, 0)),
            pl.BlockSpec((_BM1, _HH), lambda i: (i, 0)),
            pl.BlockSpec((_BM1, _OUT), lambda i: (i, 0)),
        ],
        out_shape=[
            jax.ShapeDtypeStruct((_N, _HH), jnp.float32),
            jax.ShapeDtypeStruct((_N, _HH), jnp.float32),
            jax.ShapeDtypeStruct((_N, _OUT), jnp.float32),
        ],
    )(inp, weight1, gcn1_w, weight2)


def _stage2(g1a, g1b, gcn2_w, weight2, p0):
    return pl.pallas_call(
        _stage2_body,
        grid=(_mblocks(),),
        in_specs=[
            pl.BlockSpec((_BM, _HH), lambda i: (i, 0)),
            pl.BlockSpec((_BM, _HH), lambda i: (i, 0)),
            pl.BlockSpec((_H, _H), lambda i: (0, 0)),
            pl.BlockSpec((3 * _H, _OUT), lambda i: (0, 0)),
            pl.BlockSpec((_BM, _OUT), lambda i: (i, 0)),
        ],
        out_specs=[
            pl.BlockSpec((_BM, _HH), lambda i: (i, 0)),
            pl.BlockSpec((_BM, _HH), lambda i: (i, 0)),
            pl.BlockSpec((_BM, _OUT), lambda i: (i, 0)),
        ],
        out_shape=[
            jax.ShapeDtypeStruct((_N, _HH), jnp.float32),
            jax.ShapeDtypeStruct((_N, _HH), jnp.float32),
            jax.ShapeDtypeStruct((_N, _OUT), jnp.float32),
        ],
    )(g1a, g1b, gcn2_w, weight2, p0)


def _stage3b(p, g2a, g2b, weight2):
    return pl.pallas_call(
        _stage3b_body,
        grid=(_mblocks(),),
        in_specs=[
            pl.BlockSpec((_BM, _OUT), lambda i: (i, 0)),
            pl.BlockSpec((_BM, _HH), lambda i: (i, 0)),
            pl.BlockSpec((_BM, _HH), lambda i: (i, 0)),
            pl.BlockSpec((3 * _H, _OUT), lambda i: (0, 0)),
        ],
        out_specs=pl.BlockSpec((_BM, _OUT), lambda i: (i, 0)),
        out_shape=jax.ShapeDtypeStruct((_N, _OUT), jnp.float32),
    )(p, g2a, g2b, weight2)


def kernel(inp, edge_index, weight1, gcn1_w, gcn2_w, weight2):
    src = edge_index[0].astype(jnp.int32).reshape(_NSUB, _CHUNKS, _CK)
    dst = edge_index[1].astype(jnp.int32).reshape(_NSUB, _CHUNKS, _CK)
    zeros = jnp.zeros((_ROWS_PER_SUB, _HH), jnp.float32)

    s1a, s1b, p0 = _stage1(inp, weight1, gcn1_w, weight2)
    gnn1 = _adj_spmm_sc(s1a, s1b, src, dst, zeros)
    s2a, s2b, p = _stage2(gnn1[0, :_N], gnn1[1, :_N], gcn2_w, weight2, p0)
    gnn2 = _adj_spmm_sc(s2a, s2b, src, dst, zeros)
    return _stage3b(p, gnn2[0, :_N], gnn2[1, :_N], weight2)


# speedup vs baseline: 5.4655x; 1.0494x over previous
import functools

import jax
import jax.numpy as jnp
from jax import lax
from jax.experimental import pallas as pl
from jax.experimental.pallas import tpu as pltpu
from jax.experimental.pallas import tpu_sc as plsc

_N = 10000
_E = 160000
_H = 256
_HH = 128
_OUT = 128

_NSUB = 16
_PER_SUB_E = _E // _NSUB
_CK = 125
_CHUNKS = _PER_SUB_E // _CK
_G = 16
_NG = _CHUNKS // _G
_NPAD = 10112
_ROWS_PER_SUB = _NPAD // _NSUB

_BM = 512
_BM1 = 512


_sc_mesh = plsc.VectorSubcoreMesh(core_axis_name="c", subcore_axis_name="s")


@functools.partial(
    pl.kernel,
    out_type=jax.ShapeDtypeStruct((2, _NPAD, _HH), jnp.float32),
    mesh=_sc_mesh,
    scratch_types=[
        pltpu.VMEM((2, _G, _CK), jnp.int32),
        pltpu.VMEM((2, _G, _CK), jnp.int32),
        pltpu.VMEM((_CK, _HH), jnp.float32),
        pltpu.VMEM((_CK, _HH), jnp.float32),
        pltpu.VMEM_SHARED((_NPAD, _HH), jnp.float32),
        pltpu.SemaphoreType.DMA,
        pltpu.SemaphoreType.DMA,
        pltpu.SemaphoreType.DMA,
        pltpu.SemaphoreType.DMA,
    ],
)
def _adj_spmm_sc(sup0, sup1, src_hbm, dst_hbm, zeros_hbm, out_hbm,
                 src_g, dst_g, rows0_v, rows1_v, acc_sh, sem0, sem1, semi,
                 semz):
    sid = lax.axis_index("s")
    cid = lax.axis_index("c")
    src_h = src_hbm.at[sid]
    dst_h = dst_hbm.at[sid]
    row0 = sid * _ROWS_PER_SUB
    cpz = pltpu.async_copy(zeros_hbm, acc_sh.at[pl.ds(row0, _ROWS_PER_SUB)],
                           semz)

    def run(sup, out_half):
        pltpu.sync_copy(src_h.at[pl.ds(0, _G)], src_g.at[0])
        pltpu.sync_copy(dst_h.at[pl.ds(0, _G)], dst_g.at[0])
        pltpu.async_copy(sup.at[src_g.at[0].at[0]], rows0_v, sem0)
        cpz.wait()
        plsc.subcore_barrier()

        for g in range(_NG):
            b = g % 2
            nb = (g + 1) % 2
            if g + 1 < _NG:
                cpi_s = pltpu.async_copy(
                    src_h.at[pl.ds((g + 1) * _G, _G)], src_g.at[nb], semi)
                cpi_d = pltpu.async_copy(
                    dst_h.at[pl.ds((g + 1) * _G, _G)], dst_g.at[nb], semi)

            def body(i, carry, b=b):
                k0 = 2 * i
                pltpu.make_async_copy(
                    sup.at[src_g.at[b].at[k0]], rows0_v, sem0).wait()
                pltpu.async_copy(
                    sup.at[src_g.at[b].at[k0 + 1]], rows1_v, sem1)
                pltpu.sync_copy(rows0_v, acc_sh.at[dst_g.at[b].at[k0]],
                                add=True)
                pltpu.make_async_copy(
                    sup.at[src_g.at[b].at[k0 + 1]], rows1_v, sem1).wait()

                @pl.when(k0 + 2 < _G)
                def _():
                    pltpu.async_copy(
                        sup.at[src_g.at[b].at[k0 + 2]], rows0_v, sem0)

                pltpu.sync_copy(rows1_v, acc_sh.at[dst_g.at[b].at[k0 + 1]],
                                add=True)
                return carry

            lax.fori_loop(0, _G // 2, body, 0)
            if g + 1 < _NG:
                cpi_s.wait()
                cpi_d.wait()
                pltpu.async_copy(sup.at[src_g.at[nb].at[0]], rows0_v, sem0)

        plsc.subcore_barrier()
        pltpu.sync_copy(acc_sh.at[pl.ds(row0, _ROWS_PER_SUB)],
                        out_half.at[pl.ds(row0, _ROWS_PER_SUB)])

    @pl.when(cid == 0)
    def _():
        run(sup0, out_hbm.at[0])

    @pl.when(cid == 1)
    def _():
        run(sup1, out_hbm.at[1])


def _stage1_body(a_ref, w1_ref, g1_ref, w2_ref, s1a_ref, s1b_ref, p0_ref):
    x = jnp.dot(a_ref[...], w1_ref[...], preferred_element_type=jnp.float32)
    g1 = g1_ref[...]
    s1a_ref[...] = jnp.dot(x, g1[:, :_HH], preferred_element_type=jnp.float32)
    s1b_ref[...] = jnp.dot(x, g1[:, _HH:], preferred_element_type=jnp.float32)
    p0_ref[...] = jnp.dot(x, w2_ref[...][:_H],
                          preferred_element_type=jnp.float32)


def _stage2_body(ga_ref, gb_ref, g2_ref, w2_ref, p0_ref, s2a_ref, s2b_ref,
                 p_ref):
    ga = jnp.tanh(ga_ref[...])
    gb = jnp.tanh(gb_ref[...])
    g2 = g2_ref[...]
    w2 = w2_ref[...]
    s2a_ref[...] = (jnp.dot(ga, g2[:_HH, :_HH], preferred_element_type=jnp.float32)
                    + jnp.dot(gb, g2[_HH:, :_HH], preferred_element_type=jnp.float32))
    s2b_ref[...] = (jnp.dot(ga, g2[:_HH, _HH:], preferred_element_type=jnp.float32)
                    + jnp.dot(gb, g2[_HH:, _HH:], preferred_element_type=jnp.float32))
    p_ref[...] = (p0_ref[...]
                  + jnp.dot(ga, w2[_H:_H + _HH],
                            preferred_element_type=jnp.float32)
                  + jnp.dot(gb, w2[_H + _HH:2 * _H],
                            preferred_element_type=jnp.float32))


def _stage3b_body(p_ref, g2a_ref, g2b_ref, w2_ref, o_ref):
    w2 = w2_ref[...]
    acc = p_ref[...]
    acc = acc + jnp.dot(g2a_ref[...], w2[2 * _H:2 * _H + _HH],
                        preferred_element_type=jnp.float32)
    acc = acc + jnp.dot(g2b_ref[...], w2[2 * _H + _HH:],
                        preferred_element_type=jnp.float32)
    o_ref[...] = acc


def _mblocks():
    return (_N + _BM - 1) // _BM


def _stage1(inp, weight1, gcn1_w, weight2):
    return pl.pallas_call(
        _stage1_body,
        grid=((_N + _BM1 - 1) // _BM1,),
        in_specs=[
            pl.BlockSpec((_BM1, _N), lambda i: (i, 0)),
            pl.BlockSpec((_N, _H), lambda i: (0, 0)),
            pl.BlockSpec((_H, _H), lambda i: (0, 0)),
            pl.BlockSpec((3 * _H, _OUT), lambda i: (0, 0)),
        ],
        out_specs=[
            pl.BlockSpec((_BM1, _HH), lambda i: (i, 0)),
            pl.BlockSpec((_BM1, _HH), lambda i: (i, 0)),
            pl.BlockSpec((_BM1, _OUT), lambda i: (i, 0)),
        ],
        out_shape=[
            jax.ShapeDtypeStruct((_N, _HH), jnp.float32),
            jax.ShapeDtypeStruct((_N, _HH), jnp.float32),
            jax.ShapeDtypeStruct((_N, _OUT), jnp.float32),
        ],
    )(inp, weight1, gcn1_w, weight2)


def _stage2(g1a, g1b, gcn2_w, weight2, p0):
    return pl.pallas_call(
        _stage2_body,
        grid=(_mblocks(),),
        in_specs=[
            pl.BlockSpec((_BM, _HH), lambda i: (i, 0)),
            pl.BlockSpec((_BM, _HH), lambda i: (i, 0)),
            pl.BlockSpec((_H, _H), lambda i: (0, 0)),
            pl.BlockSpec((3 * _H, _OUT), lambda i: (0, 0)),
            pl.BlockSpec((_BM, _OUT), lambda i: (i, 0)),
        ],
        out_specs=[
            pl.BlockSpec((_BM, _HH), lambda i: (i, 0)),
            pl.BlockSpec((_BM, _HH), lambda i: (i, 0)),
            pl.BlockSpec((_BM, _OUT), lambda i: (i, 0)),
        ],
        out_shape=[
            jax.ShapeDtypeStruct((_N, _HH), jnp.float32),
            jax.ShapeDtypeStruct((_N, _HH), jnp.float32),
            jax.ShapeDtypeStruct((_N, _OUT), jnp.float32),
        ],
    )(g1a, g1b, gcn2_w, weight2, p0)


def _stage3b(p, g2a, g2b, weight2):
    return pl.pallas_call(
        _stage3b_body,
        grid=(_mblocks(),),
        in_specs=[
            pl.BlockSpec((_BM, _OUT), lambda i: (i, 0)),
            pl.BlockSpec((_BM, _HH), lambda i: (i, 0)),
            pl.BlockSpec((_BM, _HH), lambda i: (i, 0)),
            pl.BlockSpec((3 * _H, _OUT), lambda i: (0, 0)),
        ],
        out_specs=pl.BlockSpec((_BM, _OUT), lambda i: (i, 0)),
        out_shape=jax.ShapeDtypeStruct((_N, _OUT), jnp.float32),
    )(p, g2a, g2b, weight2)


def kernel(inp, edge_index, weight1, gcn1_w, gcn2_w, weight2):
    src = edge_index[0].astype(jnp.int32).reshape(_NSUB, _CHUNKS, _CK)
    dst = edge_index[1].astype(jnp.int32).reshape(_NSUB, _CHUNKS, _CK)
    zeros = jnp.zeros((_ROWS_PER_SUB, _HH), jnp.float32)

    s1a, s1b, p0 = _stage1(inp, weight1, gcn1_w, weight2)
    gnn1 = _adj_spmm_sc(s1a, s1b, src, dst, zeros)
    s2a, s2b, p = _stage2(gnn1[0, :_N], gnn1[1, :_N], gcn2_w, weight2, p0)
    gnn2 = _adj_spmm_sc(s2a, s2b, src, dst, zeros)
    return _stage3b(p, gnn2[0, :_N], gnn2[1, :_N], weight2)
